# baseline (speedup 1.0000x reference)
import functools

import jax
import jax.numpy as jnp
from jax import lax
from jax.experimental import pallas as pl
from jax.experimental.pallas import tpu as pltpu
from jax.experimental.pallas import tpu_sc as plsc

_NUM_EMB = 1024
_DIM = 64
_HW = 1024
_IMGS = 16
_ROWS = _IMGS * _HW
_COMMITMENT = 0.25

_NC, _NS, _LANES = 2, 16, 16
_NW = _NC * _NS
_RPW = _ROWS // _NW
_ZROWS = 64
_ZWORDS = _ZROWS * _NUM_EMB


def _vq_body(x_ref, xsq_ref, emb_ref, embt_ref, esq_ref,
             idx_ref, q_ref, loss_ref):
    step = pl.program_id(0)
    x = x_ref[0]
    mt = jax.lax.dot_general(emb_ref[...], x,
                             (((1,), (0,)), ((), ())),
                             preferred_element_type=jnp.float32)
    dt = (xsq_ref[0] + esq_ref[...]) - 2.0 * mt
    dmin = jnp.min(dt, axis=0, keepdims=True)
    iota = jax.lax.broadcasted_iota(jnp.int32, dt.shape, 0).astype(jnp.float32)
    cand = jnp.where(dt == dmin, iota, float(_NUM_EMB))
    idx = jnp.min(cand, axis=0, keepdims=True)
    idx_ref[0] = idx.astype(jnp.int32)
    onehot_t = jnp.where(cand == idx, 1.0, 0.0)
    q = jnp.dot(embt_ref[...], onehot_t,
                preferred_element_type=jnp.float32)
    q_ref[0] = x + (q - x)

    @pl.when(step == 0)
    def _():
        loss_ref[...] = jnp.zeros_like(loss_ref)

    loss_ref[...] += jnp.sum(dmin).reshape(1, 1)


_sc_mesh = plsc.VectorSubcoreMesh(core_axis_name="c", subcore_axis_name="s")


@functools.partial(
    pl.kernel, mesh=_sc_mesh,
    out_type=jax.ShapeDtypeStruct((_ROWS * _NUM_EMB,), jnp.float32),
    scratch_types=[
        pltpu.VMEM((_ZWORDS,), jnp.float32),
        pltpu.VMEM((_RPW,), jnp.int32),
        pltpu.VMEM((_RPW // 128, 128), jnp.int32),
        pltpu.VMEM((128,), jnp.float32),
        pltpu.SemaphoreType.DMA,
        pltpu.SemaphoreType.DMA,
    ],
)
def _sc_onehot(idx_hbm, enc_ref, zbuf, idx_v, pos_v, ones_v, zsem, ssem):
    wid = lax.axis_index("s") * _NC + lax.axis_index("c")
    base = wid * _RPW
    ebase = base * _NUM_EMB
    pltpu.sync_copy(idx_hbm.at[pl.ds(base, _RPW)], idx_v)
    lane = lax.broadcasted_iota(jnp.int32, (_LANES,), 0)

    def _zinit(i):
        zbuf[pl.ds(i * _LANES, _LANES)] = jnp.zeros((_LANES,), jnp.float32)

    pl.loop(0, _ZWORDS // _LANES)(_zinit)

    nblk = (_RPW * _NUM_EMB) // _ZWORDS
    zcopies = [
        pltpu.async_copy(zbuf, enc_ref.at[pl.ds(ebase + b * _ZWORDS, _ZWORDS)],
                         zsem)
        for b in range(nblk)
    ]

    def _mkpos(k):
        row = base + k * _LANES + lane
        pos_v[k // 8, pl.ds((k % 8) * _LANES, _LANES)] = (
            row * _NUM_EMB + idx_v[pl.ds(k * _LANES, _LANES)])

    pl.loop(0, _RPW // _LANES)(_mkpos)

    def _ones(k):
        ones_v[pl.ds(k * _LANES, _LANES)] = jnp.ones((_LANES,), jnp.float32)

    pl.loop(0, 128 // _LANES)(_ones)

    for cp in zcopies:
        cp.wait()
    scopies = [
        pltpu.async_copy(ones_v, enc_ref.at[pos_v.at[j]], ssem)
        for j in range(_RPW // 128)
    ]
    for cp in scopies:
        cp.wait()


def kernel(inputs, embedding):
    x_chw = inputs.astype(jnp.float32).reshape(_IMGS, _DIM, _HW)
    emb = embedding.astype(jnp.float32)
    flat = jnp.transpose(inputs, (0, 2, 3, 1)).reshape(-1, _DIM)
    flat = flat.astype(jnp.float32)
    xsq = jnp.sum(flat ** 2, axis=1).reshape(_IMGS, 1, _HW)
    esq = jnp.sum(emb ** 2, axis=1)[:, None]
    embt = emb.T

    idx, q, loss_sum = pl.pallas_call(
        _vq_body,
        grid=(_IMGS,),
        in_specs=[
            pl.BlockSpec((1, _DIM, _HW), lambda i: (i, 0, 0)),
            pl.BlockSpec((1, 1, _HW), lambda i: (i, 0, 0)),
            pl.BlockSpec((_NUM_EMB, _DIM), lambda i: (0, 0)),
            pl.BlockSpec((_DIM, _NUM_EMB), lambda i: (0, 0)),
            pl.BlockSpec((_NUM_EMB, 1), lambda i: (0, 0)),
        ],
        out_specs=[
            pl.BlockSpec((1, 1, _HW), lambda i: (i, 0, 0)),
            pl.BlockSpec((1, _DIM, _HW), lambda i: (i, 0, 0)),
            pl.BlockSpec((1, 1), lambda i: (0, 0)),
        ],
        out_shape=[
            jax.ShapeDtypeStruct((_IMGS, 1, _HW), jnp.int32),
            jax.ShapeDtypeStruct((_IMGS, _DIM, _HW), jnp.float32),
            jax.ShapeDtypeStruct((1, 1), jnp.float32),
        ],
    )(x_chw, xsq, emb, embt, esq)

    enc = _sc_onehot(idx.reshape(_ROWS))

    quantized = q.reshape(inputs.shape)
    loss = _COMMITMENT * (loss_sum[0, 0] / (_ROWS * _DIM))
    return (quantized, loss, enc.reshape(_ROWS, _NUM_EMB))


# speedup vs baseline: 2.7881x; 2.7881x over previous
import jax
import jax.numpy as jnp
from jax.experimental import pallas as pl

_NUM_EMB = 1024
_DIM = 64
_HW = 1024
_IMGS = 16
_ROWS = _IMGS * _HW
_COMMITMENT = 0.25


def _vq_body(x_ref, xsq_ref, emb_ref, embt_ref, esq_ref,
             enc_ref, q_ref, loss_ref):
    step = pl.program_id(0)
    x = x_ref[0]
    mt = jax.lax.dot_general(emb_ref[...], x,
                             (((1,), (0,)), ((), ())),
                             preferred_element_type=jnp.float32)
    dt = (xsq_ref[0] + esq_ref[...]) - 2.0 * mt
    dmin = jnp.min(dt, axis=0, keepdims=True)
    iota = jax.lax.broadcasted_iota(jnp.int32, dt.shape, 0).astype(jnp.float32)
    idx = jnp.min(jnp.where(dt == dmin, iota, float(_NUM_EMB)),
                  axis=0, keepdims=True)
    onehot_t = jnp.where(iota == idx, 1.0, 0.0)
    enc_ref[...] = onehot_t.T
    q = jnp.dot(embt_ref[...], onehot_t,
                preferred_element_type=jnp.float32)
    q_ref[0] = x + (q - x)

    @pl.when(step == 0)
    def _():
        loss_ref[...] = jnp.zeros_like(loss_ref)

    loss_ref[...] += jnp.sum(dmin).reshape(1, 1)


def kernel(inputs, embedding):
    x_chw = inputs.astype(jnp.float32).reshape(_IMGS, _DIM, _HW)
    emb = embedding.astype(jnp.float32)
    flat = jnp.transpose(inputs, (0, 2, 3, 1)).reshape(-1, _DIM)
    flat = flat.astype(jnp.float32)
    xsq = jnp.sum(flat ** 2, axis=1).reshape(_IMGS, 1, _HW)
    esq = jnp.sum(emb ** 2, axis=1)[:, None]
    embt = emb.T

    enc, q, loss_sum = pl.pallas_call(
        _vq_body,
        grid=(_IMGS,),
        in_specs=[
            pl.BlockSpec((1, _DIM, _HW), lambda i: (i, 0, 0)),
            pl.BlockSpec((1, 1, _HW), lambda i: (i, 0, 0)),
            pl.BlockSpec((_NUM_EMB, _DIM), lambda i: (0, 0)),
            pl.BlockSpec((_DIM, _NUM_EMB), lambda i: (0, 0)),
            pl.BlockSpec((_NUM_EMB, 1), lambda i: (0, 0)),
        ],
        out_specs=[
            pl.BlockSpec((_HW, _NUM_EMB), lambda i: (i, 0)),
            pl.BlockSpec((1, _DIM, _HW), lambda i: (i, 0, 0)),
            pl.BlockSpec((1, 1), lambda i: (0, 0)),
        ],
        out_shape=[
            jax.ShapeDtypeStruct((_ROWS, _NUM_EMB), jnp.float32),
            jax.ShapeDtypeStruct((_IMGS, _DIM, _HW), jnp.float32),
            jax.ShapeDtypeStruct((1, 1), jnp.float32),
        ],
    )(x_chw, xsq, emb, embt, esq)

    quantized = q.reshape(inputs.shape)
    loss = _COMMITMENT * (loss_sum[0, 0] / (_ROWS * _DIM))
    return (quantized, loss, enc)


# speedup vs baseline: 2.8406x; 1.0188x over previous
import jax
import jax.numpy as jnp
from jax.experimental import pallas as pl

_NUM_EMB = 1024
_DIM = 64
_HW = 1024
_IMGS = 16
_ROWS = _IMGS * _HW
_COMMITMENT = 0.25


def _vq_body(x_ref, xsq_ref, emb_ref, embt_ref, esq_ref,
             enc_ref, q_ref, loss_ref):
    step = pl.program_id(0)
    x = x_ref[0]
    mt = jax.lax.dot_general(emb_ref[...], x,
                             (((1,), (0,)), ((), ())),
                             preferred_element_type=jnp.float32)
    dt = (xsq_ref[0] + esq_ref[...]) - 2.0 * mt
    dmin = jnp.min(dt, axis=0, keepdims=True)
    iota = jax.lax.broadcasted_iota(jnp.int32, dt.shape, 0).astype(jnp.float32)
    idx = jnp.min(jnp.where(dt == dmin, iota, float(_NUM_EMB)),
                  axis=0, keepdims=True)
    onehot_t = jnp.where(iota == idx, 1.0, 0.0)
    enc_ref[...] = onehot_t.T
    q = jnp.dot(embt_ref[...], onehot_t,
                preferred_element_type=jnp.float32)
    q_ref[0] = x + (q - x)

    @pl.when(step == 0)
    def _():
        loss_ref[...] = jnp.zeros_like(loss_ref)

    loss_ref[...] += jnp.sum(dmin).reshape(1, 1)

    @pl.when(step == _IMGS - 1)
    def _():
        loss_ref[...] *= _COMMITMENT / (_ROWS * _DIM)


def kernel(inputs, embedding):
    x_chw = inputs.astype(jnp.float32).reshape(_IMGS, _DIM, _HW)
    emb = embedding.astype(jnp.float32)
    flat = jnp.transpose(inputs, (0, 2, 3, 1)).reshape(-1, _DIM)
    flat = flat.astype(jnp.float32)
    xsq = jnp.sum(flat ** 2, axis=1).reshape(_IMGS, 1, _HW)
    esq = jnp.sum(emb ** 2, axis=1)[:, None]
    embt = emb.T

    enc, q, loss_sum = pl.pallas_call(
        _vq_body,
        grid=(_IMGS,),
        in_specs=[
            pl.BlockSpec((1, _DIM, _HW), lambda i: (i, 0, 0)),
            pl.BlockSpec((1, 1, _HW), lambda i: (i, 0, 0)),
            pl.BlockSpec((_NUM_EMB, _DIM), lambda i: (0, 0)),
            pl.BlockSpec((_DIM, _NUM_EMB), lambda i: (0, 0)),
            pl.BlockSpec((_NUM_EMB, 1), lambda i: (0, 0)),
        ],
        out_specs=[
            pl.BlockSpec((_HW, _NUM_EMB), lambda i: (i, 0)),
            pl.BlockSpec((1, _DIM, _HW), lambda i: (i, 0, 0)),
            pl.BlockSpec((1, 1), lambda i: (0, 0)),
        ],
        out_shape=[
            jax.ShapeDtypeStruct((_ROWS, _NUM_EMB), jnp.float32),
            jax.ShapeDtypeStruct((_IMGS, _DIM, _HW), jnp.float32),
            jax.ShapeDtypeStruct((1, 1), jnp.float32),
        ],
    )(x_chw, xsq, emb, embt, esq)

    quantized = q.reshape(inputs.shape)
    return (quantized, loss_sum[0, 0], enc)
